# baseline (speedup 1.0000x reference)
import jax
import jax.numpy as jnp
from jax import lax
from jax.experimental import pallas as pl
from jax.experimental.pallas import tpu as pltpu
from jax.experimental.pallas import tpu_sc as plsc

NUM_LEVELS = 16
LOG2_T = 19
TBL = 1 << LOG2_T
N_PTS = 1048576
HIDDEN = 64
ENC_D = 2 * NUM_LEVELS

_RESOLUTIONS = [16, 20, 25, 32, 40, 50, 64, 80, 101, 128, 161, 203, 256, 322, 406, 512]
_P2 = -1640531535
_P3 = 805459861
_MASK = TBL - 1

NC = 2
NS = 16
NW = NC * NS
PTS_PER_W = N_PTS // NW
CHUNK = 1024
NCHUNKS = PTS_PER_W // CHUNK
NGROUPS = CHUNK // 16


def _encode_body(posf, tabf, out,
                 psv, xs, ys, zs, idx0, idx1, rows0, rows1, encf,
                 semA, semB):
    wid = lax.axis_index("s") * NC + lax.axis_index("c")
    base = wid * PTS_PER_W
    iota = lax.iota(jnp.int32, 16)
    iota3 = iota * 3
    sems = (semA, semB)

    def compute_idx(l, b):
        res = float(_RESOLUTIONS[l])
        lvl_off2 = 2 * l * TBL
        i0v, i1v = idx0[b], idx1[b]

        def idx_group(g, carry_):
            s = g * 16
            x = xs[pl.ds(s, 16)] * res
            y = ys[pl.ds(s, 16)] * res
            z = zs[pl.ds(s, 16)] * res
            xi = x.astype(jnp.int32)
            yi = y.astype(jnp.int32)
            zi = z.astype(jnp.int32)
            yh0 = yi * _P2
            yh1 = yh0 + _P2
            zh0 = zi * _P3
            zh1 = zh0 + _P3
            xi1 = xi + 1
            e = (yh0 ^ zh0, yh0 ^ zh1, yh1 ^ zh0, yh1 ^ zh1)
            j = 0
            for xv in (xi, xi1):
                for yz in range(4):
                    i0 = (((xv ^ e[yz]) & _MASK) << 1) + lvl_off2
                    i0v[pl.ds(j * CHUNK + s, 16)] = i0
                    i1v[pl.ds(j * CHUNK + s, 16)] = i0 + 1
                    j += 1
            return carry_

        lax.fori_loop(0, NGROUPS, idx_group, 0)

    def fire(b):
        return (pltpu.async_copy(tabf.at[idx0[b]], rows0[b], sems[b]),
                pltpu.async_copy(tabf.at[idx1[b]], rows1[b], sems[b]))

    def interp(l, b):
        res = float(_RESOLUTIONS[l])
        r0, r1 = rows0[b], rows1[b]

        def interp_group(g, carry_):
            s = g * 16
            x = xs[pl.ds(s, 16)] * res
            y = ys[pl.ds(s, 16)] * res
            z = zs[pl.ds(s, 16)] * res
            wx = x - x.astype(jnp.int32).astype(jnp.float32)
            wy = y - y.astype(jnp.int32).astype(jnp.float32)
            wz = z - z.astype(jnp.int32).astype(jnp.float32)
            sid = (s + iota) * ENC_D + (2 * l)
            for ft, rows in ((0, r0), (1, r1)):
                f = [rows[pl.ds(j * CHUNK + s, 16)] for j in range(8)]
                c00 = f[0] + wz * (f[1] - f[0])
                c01 = f[2] + wz * (f[3] - f[2])
                c10 = f[4] + wz * (f[5] - f[4])
                c11 = f[6] + wz * (f[7] - f[6])
                d0 = c00 + wy * (c01 - c00)
                d1 = c10 + wy * (c11 - c10)
                v = d0 + wx * (d1 - d0)
                plsc.store_scatter(encf, [sid + ft], v)
            return carry_

        lax.fori_loop(0, NGROUPS, interp_group, 0)

    def chunk_body(c, carry):
        cbase = base + c * CHUNK
        pltpu.sync_copy(posf.at[pl.ds(cbase * 3, CHUNK * 3)], psv)

        def deint_group(g, carry_):
            s = g * 16
            bi = (s * 3) + iota3
            xs[pl.ds(s, 16)] = plsc.load_gather(psv, [bi])
            ys[pl.ds(s, 16)] = plsc.load_gather(psv, [bi + 1])
            zs[pl.ds(s, 16)] = plsc.load_gather(psv, [bi + 2])
            return carry_

        lax.fori_loop(0, NGROUPS, deint_group, 0)

        compute_idx(0, 0)
        pend = fire(0)
        for l in range(NUM_LEVELS):
            b = l & 1
            if l + 1 < NUM_LEVELS:
                compute_idx(l + 1, 1 - b)
            pend[0].wait()
            pend[1].wait()
            if l + 1 < NUM_LEVELS:
                pend = fire(1 - b)
            interp(l, b)

        pltpu.sync_copy(encf, out.at[pl.ds(cbase * ENC_D, CHUNK * ENC_D)])
        return carry

    lax.fori_loop(0, NCHUNKS, chunk_body, 0)


def _hash_encode_sc(posf, tabf):
    mesh = plsc.VectorSubcoreMesh(core_axis_name="c", subcore_axis_name="s",
                                  num_cores=NC, num_subcores=NS)

    def body(posf, tabf, out, psv, xs, ys, zs,
             i0a, i0b, i1a, i1b, r0a, r0b, r1a, r1b, encf, semA, semB):
        _encode_body(posf, tabf, out, psv, xs, ys, zs,
                     (i0a, i0b), (i1a, i1b), (r0a, r0b), (r1a, r1b),
                     encf, semA, semB)

    f = pl.kernel(
        body,
        out_type=jax.ShapeDtypeStruct((N_PTS * ENC_D,), jnp.float32),
        mesh=mesh,
        scratch_types=(
            [pltpu.VMEM((CHUNK * 3,), jnp.float32)]
            + [pltpu.VMEM((CHUNK,), jnp.float32)] * 3
            + [pltpu.VMEM((8 * CHUNK,), jnp.int32)] * 4
            + [pltpu.VMEM((8 * CHUNK,), jnp.float32)] * 4
            + [pltpu.VMEM((CHUNK * ENC_D,), jnp.float32)]
            + [pltpu.SemaphoreType.DMA, pltpu.SemaphoreType.DMA]
        ),
        compiler_params=pltpu.CompilerParams(needs_layout_passes=False),
    )
    return f(posf, tabf)


def _mlp_body(enc_ref, d_ref, w1_ref, b1_ref, w2_ref, b2_ref,
              w3_ref, b3_ref, rgb_ref, den_ref):
    d = d_ref[...]
    nrm = jnp.sqrt(jnp.sum(d * d, axis=1, keepdims=True))
    dn = d / jnp.maximum(nrm, 1e-12)
    bf = jnp.bfloat16
    mm = lambda a, w: jnp.dot(a.astype(bf), w.astype(bf),
                              preferred_element_type=jnp.float32)
    x = jnp.concatenate([enc_ref[...], dn], axis=-1)
    h = jnp.maximum(mm(x, w1_ref[...]) + b1_ref[...], 0.0)
    h = jnp.maximum(mm(h, w2_ref[...]) + b2_ref[...], 0.0)
    o = mm(h, w3_ref[...]) + b3_ref[...]
    rgb_ref[...] = jax.nn.sigmoid(o[:, 0:3])
    den_ref[...] = jnp.maximum(o[:, 3:4], 0.0)


def _mlp_tc(enc, directions, W1, b1, W2, b2, W3, b3):
    B = 2048
    grid = (N_PTS // B,)
    w1p = jnp.pad(W1, ((0, 5), (0, 0)))
    rep = lambda i: (0, 0)
    return pl.pallas_call(
        _mlp_body,
        grid=grid,
        in_specs=[
            pl.BlockSpec((B, 32), lambda i: (i, 0)),
            pl.BlockSpec((B, 8), lambda i: (i, 0)),
            pl.BlockSpec((40, HIDDEN), rep),
            pl.BlockSpec((1, HIDDEN), rep),
            pl.BlockSpec((HIDDEN, HIDDEN), rep),
            pl.BlockSpec((1, HIDDEN), rep),
            pl.BlockSpec((HIDDEN, 4), rep),
            pl.BlockSpec((1, 4), rep),
        ],
        out_specs=[
            pl.BlockSpec((B, 3), lambda i: (i, 0)),
            pl.BlockSpec((B, 1), lambda i: (i, 0)),
        ],
        out_shape=[
            jax.ShapeDtypeStruct((N_PTS, 3), jnp.float32),
            jax.ShapeDtypeStruct((N_PTS, 1), jnp.float32),
        ],
    )(enc, jnp.pad(directions, ((0, 0), (0, 5))), w1p, b1.reshape(1, HIDDEN),
      W2, b2.reshape(1, HIDDEN), W3, b3.reshape(1, 4))


def kernel(positions, directions, hash_tables, W1, b1, W2, b2, W3, b3):
    posf = positions.reshape(N_PTS * 3)
    tabf = hash_tables.reshape(NUM_LEVELS * TBL * 2)
    enc = _hash_encode_sc(posf, tabf)
    enc = enc.reshape(N_PTS, ENC_D)
    rgb, density = _mlp_tc(enc, directions, W1, b1, W2, b2, W3, b3)
    return (rgb, density)


# speedup vs baseline: 3.1485x; 3.1485x over previous
import jax
import jax.numpy as jnp
from jax import lax
from jax.experimental import pallas as pl
from jax.experimental.pallas import tpu as pltpu
from jax.experimental.pallas import tpu_sc as plsc

NUM_LEVELS = 16
LOG2_T = 19
TBL = 1 << LOG2_T
N_PTS = 1048576
HIDDEN = 64
ENC_D = 2 * NUM_LEVELS

_RESOLUTIONS = [16, 20, 25, 32, 40, 50, 64, 80, 101, 128, 161, 203, 256, 322, 406, 512]
_P2 = -1640531535
_P3 = 805459861
_MASK = TBL - 1

NC = 2
NS = 16
NW = NC * NS
PTS_PER_W = N_PTS // NW
CHUNK = 512
NCHUNKS = PTS_PER_W // CHUNK
NGROUPS = CHUNK // 16
IDXN = 8 * CHUNK

N_DENSE = 4
_DENSE_BATCHES = [(r + 1) ** 3 // IDXN + 1 for r in _RESOLUTIONS[:N_DENSE]]
_DENSE_OFF = [sum(b * IDXN for b in _DENSE_BATCHES[:i]) for i in range(N_DENSE)]
DENSE_WORDS = sum(b * IDXN for b in _DENSE_BATCHES)


def _encode_body(xp, yp, zp, tabp, out,
                 xs, ys, zs, idx0, rows0, dense, encf,
                 semA, semB):
    wid = lax.axis_index("s") * NC + lax.axis_index("c")
    base = wid * PTS_PER_W
    iota = lax.iota(jnp.int32, 16)
    sems = (semA, semB)

    for l in range(N_DENSE):
        r1 = _RESOLUTIONS[l] + 1
        lvl_off = l * TBL
        for bt in range(_DENSE_BATCHES[l]):
            vbase = bt * IDXN

            @plsc.parallel_loop(0, IDXN // 16, unroll=2)
            def dense_idx_group(g, vbase=vbase, r1=r1, lvl_off=lvl_off):
                vid = vbase + g * 16 + iota
                zi = vid % r1
                t = vid // r1
                yi = t % r1
                xi = t // r1
                h = xi ^ (yi * _P2) ^ (zi * _P3)
                idx0[0][pl.ds(g * 16, 16)] = (h & _MASK) + lvl_off

            pltpu.async_copy(
                tabp.at[idx0[0]],
                dense.at[pl.ds(_DENSE_OFF[l] + vbase, IDXN)], semA).wait()

    def interp_dense(l):
        res = float(_RESOLUTIONS[l])
        r1 = _RESOLUTIONS[l] + 1
        doff = _DENSE_OFF[l]

        @plsc.parallel_loop(0, NGROUPS, unroll=2)
        def interp_group(g):
            s = g * 16
            x = xs[pl.ds(s, 16)] * res
            y = ys[pl.ds(s, 16)] * res
            z = zs[pl.ds(s, 16)] * res
            xi = x.astype(jnp.int32)
            yi = y.astype(jnp.int32)
            zi = z.astype(jnp.int32)
            wx = x - xi.astype(jnp.float32)
            wy = y - yi.astype(jnp.float32)
            wz = z - zi.astype(jnp.float32)
            vb = (xi * r1 + yi) * r1 + zi + doff
            sid = (s + iota) * ENC_D + (2 * l)
            f0 = []
            f1 = []
            for dx in (0, r1 * r1):
                for dy in (0, r1):
                    for dz in (0, 1):
                        pk = plsc.load_gather(dense, [vb + (dx + dy + dz)])
                        a, bv = plsc.unpack(
                            plsc.bitcast(pk, jnp.bfloat16),
                            format=plsc.PackFormat.INTERLEAVED,
                            preferred_element_type=jnp.float32)
                        f0.append(a)
                        f1.append(bv)
            for ft, f in ((0, f0), (1, f1)):
                c00 = f[0] + wz * (f[1] - f[0])
                c01 = f[2] + wz * (f[3] - f[2])
                c10 = f[4] + wz * (f[5] - f[4])
                c11 = f[6] + wz * (f[7] - f[6])
                d0 = c00 + wy * (c01 - c00)
                d1 = c10 + wy * (c11 - c10)
                v = d0 + wx * (d1 - d0)
                plsc.store_scatter(encf, [sid + ft], v)

    def compute_idx(l, b):
        res = float(_RESOLUTIONS[l])
        lvl_off = l * TBL
        i0v = idx0[b]

        @plsc.parallel_loop(0, NGROUPS, unroll=2)
        def idx_group(g):
            s = g * 16
            x = xs[pl.ds(s, 16)] * res
            y = ys[pl.ds(s, 16)] * res
            z = zs[pl.ds(s, 16)] * res
            xi = x.astype(jnp.int32)
            yi = y.astype(jnp.int32)
            zi = z.astype(jnp.int32)
            yh0 = yi * _P2
            yh1 = yh0 + _P2
            zh0 = zi * _P3
            zh1 = zh0 + _P3
            xi1 = xi + 1
            e = (yh0 ^ zh0, yh0 ^ zh1, yh1 ^ zh0, yh1 ^ zh1)
            j = 0
            for xv in (xi, xi1):
                for yz in range(4):
                    i0v[pl.ds(j * CHUNK + s, 16)] = ((xv ^ e[yz]) & _MASK) + lvl_off
                    j += 1

    def fire(b):
        return (pltpu.async_copy(tabp.at[idx0[b]], rows0[b], sems[b]),)

    def interp(l, b):
        res = float(_RESOLUTIONS[l])
        r0 = rows0[b]

        @plsc.parallel_loop(0, NGROUPS, unroll=2)
        def interp_group(g):
            s = g * 16
            x = xs[pl.ds(s, 16)] * res
            y = ys[pl.ds(s, 16)] * res
            z = zs[pl.ds(s, 16)] * res
            wx = x - x.astype(jnp.int32).astype(jnp.float32)
            wy = y - y.astype(jnp.int32).astype(jnp.float32)
            wz = z - z.astype(jnp.int32).astype(jnp.float32)
            sid = (s + iota) * ENC_D + (2 * l)
            f0 = []
            f1 = []
            for j in range(8):
                pk = plsc.bitcast(r0[pl.ds(j * CHUNK + s, 16)], jnp.bfloat16)
                a, bv = plsc.unpack(pk, format=plsc.PackFormat.INTERLEAVED,
                                    preferred_element_type=jnp.float32)
                f0.append(a)
                f1.append(bv)
            for ft, f in ((0, f0), (1, f1)):
                c00 = f[0] + wz * (f[1] - f[0])
                c01 = f[2] + wz * (f[3] - f[2])
                c10 = f[4] + wz * (f[5] - f[4])
                c11 = f[6] + wz * (f[7] - f[6])
                d0 = c00 + wy * (c01 - c00)
                d1 = c10 + wy * (c11 - c10)
                v = d0 + wx * (d1 - d0)
                plsc.store_scatter(encf, [sid + ft], v)

    def chunk_body(c, carry):
        cbase = base + c * CHUNK
        pltpu.sync_copy(xp.at[pl.ds(cbase, CHUNK)], xs)
        pltpu.sync_copy(yp.at[pl.ds(cbase, CHUNK)], ys)
        pltpu.sync_copy(zp.at[pl.ds(cbase, CHUNK)], zs)

        compute_idx(N_DENSE, 0)
        pend = fire(0)
        for l in range(N_DENSE):
            interp_dense(l)
        for l in range(N_DENSE, NUM_LEVELS):
            b = (l - N_DENSE) & 1
            if l + 1 < NUM_LEVELS:
                compute_idx(l + 1, 1 - b)
            for p in pend:
                p.wait()
            if l + 1 < NUM_LEVELS:
                pend = fire(1 - b)
            interp(l, b)

        pltpu.sync_copy(encf, out.at[pl.ds(cbase * ENC_D, CHUNK * ENC_D)])
        return carry

    lax.fori_loop(0, NCHUNKS, chunk_body, 0)


def _hash_encode_sc(xp, yp, zp, tabp):
    mesh = plsc.VectorSubcoreMesh(core_axis_name="c", subcore_axis_name="s",
                                  num_cores=NC, num_subcores=NS)

    def body(xp, yp, zp, tabp, out, xs, ys, zs,
             i0a, i0b, r0a, r0b, dense, encf, semA, semB):
        _encode_body(xp, yp, zp, tabp, out, xs, ys, zs,
                     (i0a, i0b), (r0a, r0b), dense,
                     encf, semA, semB)

    f = pl.kernel(
        body,
        out_type=jax.ShapeDtypeStruct((N_PTS * ENC_D,), jnp.float32),
        mesh=mesh,
        scratch_types=(
            [pltpu.VMEM((CHUNK,), jnp.float32)] * 3
            + [pltpu.VMEM((IDXN,), jnp.int32)] * 2
            + [pltpu.VMEM((IDXN,), jnp.float32)] * 2
            + [pltpu.VMEM((DENSE_WORDS,), jnp.float32)]
            + [pltpu.VMEM((CHUNK * ENC_D,), jnp.float32)]
            + [pltpu.SemaphoreType.DMA, pltpu.SemaphoreType.DMA]
        ),
        compiler_params=pltpu.CompilerParams(needs_layout_passes=False),
    )
    return f(xp, yp, zp, tabp)


def _mlp_body(enc_ref, d_ref, w1_ref, b1_ref, w2_ref, b2_ref,
              w3_ref, b3_ref, rgb_ref, den_ref):
    d = d_ref[...]
    nrm = jnp.sqrt(jnp.sum(d * d, axis=1, keepdims=True))
    dn = d / jnp.maximum(nrm, 1e-12)
    bf = jnp.bfloat16
    mm = lambda a, w: jnp.dot(a.astype(bf), w.astype(bf),
                              preferred_element_type=jnp.float32)
    x = jnp.concatenate([enc_ref[...], dn], axis=-1)
    h = jnp.maximum(mm(x, w1_ref[...]) + b1_ref[...], 0.0)
    h = jnp.maximum(mm(h, w2_ref[...]) + b2_ref[...], 0.0)
    o = mm(h, w3_ref[...]) + b3_ref[...]
    rgb_ref[...] = jax.nn.sigmoid(o[:, 0:3])
    den_ref[...] = jnp.maximum(o[:, 3:4], 0.0)


def _mlp_tc(enc, directions, W1, b1, W2, b2, W3, b3):
    B = 2048
    grid = (N_PTS // B,)
    w1p = jnp.pad(W1, ((0, 5), (0, 0)))
    rep = lambda i: (0, 0)
    return pl.pallas_call(
        _mlp_body,
        grid=grid,
        in_specs=[
            pl.BlockSpec((B, 32), lambda i: (i, 0)),
            pl.BlockSpec((B, 8), lambda i: (i, 0)),
            pl.BlockSpec((40, HIDDEN), rep),
            pl.BlockSpec((1, HIDDEN), rep),
            pl.BlockSpec((HIDDEN, HIDDEN), rep),
            pl.BlockSpec((1, HIDDEN), rep),
            pl.BlockSpec((HIDDEN, 4), rep),
            pl.BlockSpec((1, 4), rep),
        ],
        out_specs=[
            pl.BlockSpec((B, 3), lambda i: (i, 0)),
            pl.BlockSpec((B, 1), lambda i: (i, 0)),
        ],
        out_shape=[
            jax.ShapeDtypeStruct((N_PTS, 3), jnp.float32),
            jax.ShapeDtypeStruct((N_PTS, 1), jnp.float32),
        ],
    )(enc, jnp.pad(directions, ((0, 0), (0, 5))), w1p, b1.reshape(1, HIDDEN),
      W2, b2.reshape(1, HIDDEN), W3, b3.reshape(1, 4))


def kernel(positions, directions, hash_tables, W1, b1, W2, b2, W3, b3):
    pt = positions.T
    tabp = jax.lax.bitcast_convert_type(
        hash_tables.astype(jnp.bfloat16), jnp.float32).reshape(NUM_LEVELS * TBL)
    enc = _hash_encode_sc(pt[0], pt[1], pt[2], tabp)
    enc = enc.reshape(N_PTS, ENC_D)
    rgb, density = _mlp_tc(enc, directions, W1, b1, W2, b2, W3, b3)
    return (rgb, density)
